# baseline (device time: 186980 ns/iter reference)
import jax
import jax.numpy as jnp
from jax import lax
from jax.experimental import pallas as pl
from jax.experimental.pallas import tpu as pltpu

N_DEV = 8


def kernel(x, Wq, Wo, K_ext, V_ext):
    B, Sq, D = x.shape
    _, Skv, H, Dh = K_ext.shape
    scale = 1.0 / (Dh ** 0.5)

    def body(x_ref, wq_ref, wo_ref, k_ref, v_ref, out_ref,
             kv_ref, send_sems, recv_sems):
        my = lax.axis_index("i")
        left = lax.rem(my + (N_DEV - 1), N_DEV)
        right = lax.rem(my + 1, N_DEV)

        barrier_sem = pltpu.get_barrier_semaphore()
        for nbr in (left, right):
            pl.semaphore_signal(
                barrier_sem, inc=1,
                device_id=(nbr,), device_id_type=pl.DeviceIdType.MESH,
            )
        pl.semaphore_wait(barrier_sem, 2)

        kv_ref[0, 0] = k_ref[...]
        kv_ref[0, 1] = v_ref[...]

        for h in range(N_DEV - 1):
            rdma = pltpu.make_async_remote_copy(
                src_ref=kv_ref.at[h],
                dst_ref=kv_ref.at[h + 1],
                send_sem=send_sems.at[h],
                recv_sem=recv_sems.at[h],
                device_id=(right,),
                device_id_type=pl.DeviceIdType.MESH,
            )
            rdma.start()
            rdma.wait()

        q = jnp.dot(x_ref[...].reshape(B * Sq, D), wq_ref[...],
                    preferred_element_type=jnp.float32)

        k_chunks = [kv_ref[o, 0].reshape(B * Skv, H * Dh)
                    for o in range(N_DEV)]
        v_chunks = [kv_ref[o, 1].reshape(B * Skv, H * Dh)
                    for o in range(N_DEV)]

        outs = []
        for b in range(B):
            head_outs = []
            for h in range(H):
                q_bh = q[b * Sq:(b + 1) * Sq, h * Dh:(h + 1) * Dh]
                k_bh = jnp.concatenate(
                    [kc[b * Skv:(b + 1) * Skv, h * Dh:(h + 1) * Dh]
                     for kc in k_chunks], axis=0)
                v_bh = jnp.concatenate(
                    [vc[b * Skv:(b + 1) * Skv, h * Dh:(h + 1) * Dh]
                     for vc in v_chunks], axis=0)
                s = jnp.dot(q_bh, k_bh.T,
                            preferred_element_type=jnp.float32) * scale
                m = jnp.max(s, axis=1, keepdims=True)
                p = jnp.exp(s - m)
                l = jnp.sum(p, axis=1, keepdims=True)
                o_bh = jnp.dot(p, v_bh,
                               preferred_element_type=jnp.float32) / l
                head_outs.append(o_bh)
            outs.append(jnp.concatenate(head_outs, axis=1))
        attn = jnp.concatenate(outs, axis=0)

        out = jnp.dot(attn, wo_ref[...], preferred_element_type=jnp.float32)
        out_ref[...] = out.reshape(B, Sq, D)

    return pl.pallas_call(
        body,
        out_shape=jax.ShapeDtypeStruct((B, Sq, D), jnp.float32),
        in_specs=[pl.BlockSpec(memory_space=pltpu.VMEM)] * 5,
        out_specs=pl.BlockSpec(memory_space=pltpu.VMEM),
        scratch_shapes=[
            pltpu.VMEM((N_DEV, 2, B, Skv, H, Dh), jnp.float32),
            pltpu.SemaphoreType.DMA((N_DEV - 1,)),
            pltpu.SemaphoreType.DMA((N_DEV - 1,)),
        ],
        compiler_params=pltpu.CompilerParams(collective_id=0),
    )(x, Wq, Wo, K_ext, V_ext)


# device time: 111470 ns/iter; 1.6774x vs baseline; 1.6774x over previous
import jax
import jax.numpy as jnp
from jax import lax
from jax.experimental import pallas as pl
from jax.experimental.pallas import tpu as pltpu

N_DEV = 8
N_R = N_DEV // 2
N_L = N_DEV - 1 - N_R


def _nt(a, b):
    return lax.dot_general(a, b, (((1,), (1,)), ((), ())),
                           preferred_element_type=jnp.float32)


def _nn(a, b):
    return lax.dot_general(a, b, (((1,), (0,)), ((), ())),
                           preferred_element_type=jnp.float32)


def kernel(x, Wq, Wo, K_ext, V_ext):
    B, Sq, D = x.shape
    _, Skv, H, Dh = K_ext.shape
    scale = 1.0 / (Dh ** 0.5)
    BH = B * H

    def body(x_ref, wq_ref, wo_ref, k_ref, v_ref, out_ref,
             kv_ref, r_send, r_recv, l_send, l_recv):
        my = lax.axis_index("i")
        left = lax.rem(my + (N_DEV - 1), N_DEV)
        right = lax.rem(my + 1, N_DEV)

        k2 = k_ref[...].reshape(B * Skv, H * Dh)
        v2 = v_ref[...].reshape(B * Skv, H * Dh)
        for b in range(B):
            for h in range(H):
                kv_ref[0, 0, b, h] = k2[b * Skv:(b + 1) * Skv,
                                        h * Dh:(h + 1) * Dh]
                kv_ref[0, 1, b, h] = v2[b * Skv:(b + 1) * Skv,
                                        h * Dh:(h + 1) * Dh]

        barrier_sem = pltpu.get_barrier_semaphore()
        for nbr in (left, right):
            pl.semaphore_signal(
                barrier_sem, inc=1,
                device_id=(nbr,), device_id_type=pl.DeviceIdType.MESH,
            )
        pl.semaphore_wait(barrier_sem, 2)

        q_t = [None] * BH
        m_t = [None] * BH
        l_t = [None] * BH
        acc = [None] * BH

        def fold_chunk(slot):
            kc = kv_ref[slot, 0]
            vc = kv_ref[slot, 1]
            for i in range(BH):
                b, h = divmod(i, H)
                s = _nt(q_t[i], kc[b, h]) * scale
                if m_t[i] is None:
                    m_t[i] = jnp.max(s, axis=1, keepdims=True)
                    p = jnp.exp(s - m_t[i])
                    l_t[i] = jnp.sum(p, axis=1, keepdims=True)
                    acc[i] = _nn(p, vc[b, h])
                else:
                    m_new = jnp.maximum(m_t[i],
                                        jnp.max(s, axis=1, keepdims=True))
                    alpha = jnp.exp(m_t[i] - m_new)
                    p = jnp.exp(s - m_new)
                    l_t[i] = l_t[i] * alpha + jnp.sum(p, axis=1,
                                                      keepdims=True)
                    acc[i] = acc[i] * alpha + _nn(p, vc[b, h])
                    m_t[i] = m_new

        for r in range(N_R):
            rdmas = []
            if r < N_R:
                rdma = pltpu.make_async_remote_copy(
                    src_ref=kv_ref.at[r],
                    dst_ref=kv_ref.at[r + 1],
                    send_sem=r_send.at[r],
                    recv_sem=r_recv.at[r],
                    device_id=(right,),
                    device_id_type=pl.DeviceIdType.MESH,
                )
                rdma.start()
                rdmas.append(rdma)
            if r < N_L:
                src = 0 if r == 0 else N_R + r
                rdma = pltpu.make_async_remote_copy(
                    src_ref=kv_ref.at[src],
                    dst_ref=kv_ref.at[N_R + 1 + r],
                    send_sem=l_send.at[r],
                    recv_sem=l_recv.at[r],
                    device_id=(left,),
                    device_id_type=pl.DeviceIdType.MESH,
                )
                rdma.start()
                rdmas.append(rdma)

            if r == 0:
                q = jnp.dot(x_ref[...].reshape(B * Sq, D), wq_ref[...],
                            preferred_element_type=jnp.float32)
                for i in range(BH):
                    b, h = divmod(i, H)
                    q_t[i] = q[b * Sq:(b + 1) * Sq, h * Dh:(h + 1) * Dh]
                fold_chunk(0)
            else:
                fold_chunk(r)
                fold_chunk(N_R + r)

            for rdma in rdmas:
                rdma.wait()

        fold_chunk(N_R)

        attn = jnp.concatenate(
            [jnp.concatenate([acc[b * H + h] / l_t[b * H + h]
                              for h in range(H)], axis=1)
             for b in range(B)], axis=0)
        out = jnp.dot(attn, wo_ref[...], preferred_element_type=jnp.float32)
        out_ref[...] = out.reshape(B, Sq, D)

    return pl.pallas_call(
        body,
        out_shape=jax.ShapeDtypeStruct((B, Sq, D), jnp.float32),
        in_specs=[pl.BlockSpec(memory_space=pltpu.VMEM)] * 5,
        out_specs=pl.BlockSpec(memory_space=pltpu.VMEM),
        scratch_shapes=[
            pltpu.VMEM((N_DEV, 2, B, H, Skv, Dh), jnp.float32),
            pltpu.SemaphoreType.DMA((N_R,)),
            pltpu.SemaphoreType.DMA((N_R,)),
            pltpu.SemaphoreType.DMA((N_L,)),
            pltpu.SemaphoreType.DMA((N_L,)),
        ],
        compiler_params=pltpu.CompilerParams(collective_id=0),
    )(x, Wq, Wo, K_ext, V_ext)


# device time: 85076 ns/iter; 2.1978x vs baseline; 1.3102x over previous
import jax
import jax.numpy as jnp
from jax import lax
from jax.experimental import pallas as pl
from jax.experimental.pallas import tpu as pltpu

N_DEV = 8


def _nt(a, b):
    return lax.dot_general(a, b, (((1,), (1,)), ((), ())),
                           preferred_element_type=jnp.float32)


def _nn(a, b):
    return lax.dot_general(a, b, (((1,), (0,)), ((), ())),
                           preferred_element_type=jnp.float32)


def kernel(x, Wq, Wo, K_ext, V_ext):
    B, Sq, D = x.shape
    _, Skv, H, Dh = K_ext.shape
    scale = 1.0 / (Dh ** 0.5)
    BH = B * H

    def body(x_ref, wq_ref, wo_ref, k_ref, v_ref, out_ref,
             kv_ref, send_sems, recv_sems):
        my = lax.axis_index("i")
        px = jnp.bitwise_xor(my, 1)
        py = jnp.bitwise_xor(my, 3)
        pz = jnp.bitwise_xor(my, 4)

        k2 = k_ref[...].reshape(B * Skv, H * Dh)
        v2 = v_ref[...].reshape(B * Skv, H * Dh)
        for b in range(B):
            for h in range(H):
                kv_ref[0, 0, b, h] = k2[b * Skv:(b + 1) * Skv,
                                        h * Dh:(h + 1) * Dh]
                kv_ref[0, 1, b, h] = v2[b * Skv:(b + 1) * Skv,
                                        h * Dh:(h + 1) * Dh]

        barrier_sem = pltpu.get_barrier_semaphore()
        for p in (px, py, pz):
            pl.semaphore_signal(
                barrier_sem, inc=1,
                device_id=(p,), device_id_type=pl.DeviceIdType.MESH,
            )
        pl.semaphore_wait(barrier_sem, 3)

        rounds = [
            [(0, 0, 1, px), (1, 0, 2, py), (2, 0, 3, pz)],
            [(3, 2, 4, px), (4, 3, 5, py), (5, 1, 6, pz)],
            [(6, 5, 7, px)],
        ]

        q_t = [None] * BH
        m_t = [None] * BH
        l_t = [None] * BH
        acc = [None] * BH

        def fold_chunk(slot):
            kc = kv_ref[slot, 0]
            vc = kv_ref[slot, 1]
            for i in range(BH):
                b, h = divmod(i, H)
                s = _nt(q_t[i], kc[b, h]) * scale
                if m_t[i] is None:
                    m_t[i] = jnp.max(s, axis=1, keepdims=True)
                    p = jnp.exp(s - m_t[i])
                    l_t[i] = jnp.sum(p, axis=1, keepdims=True)
                    acc[i] = _nn(p, vc[b, h])
                else:
                    m_new = jnp.maximum(m_t[i],
                                        jnp.max(s, axis=1, keepdims=True))
                    alpha = jnp.exp(m_t[i] - m_new)
                    p = jnp.exp(s - m_new)
                    l_t[i] = l_t[i] * alpha + jnp.sum(p, axis=1,
                                                      keepdims=True)
                    acc[i] = acc[i] * alpha + _nn(p, vc[b, h])
                    m_t[i] = m_new

        folds = [[0], [1, 2, 3], [4, 5, 6]]

        for r, transfers in enumerate(rounds):
            rdmas = []
            for t, src, dst, partner in transfers:
                rdma = pltpu.make_async_remote_copy(
                    src_ref=kv_ref.at[src],
                    dst_ref=kv_ref.at[dst],
                    send_sem=send_sems.at[t],
                    recv_sem=recv_sems.at[t],
                    device_id=(partner,),
                    device_id_type=pl.DeviceIdType.MESH,
                )
                rdma.start()
                rdmas.append(rdma)

            if r == 0:
                q = jnp.dot(x_ref[...].reshape(B * Sq, D), wq_ref[...],
                            preferred_element_type=jnp.float32)
                for i in range(BH):
                    b, h = divmod(i, H)
                    q_t[i] = q[b * Sq:(b + 1) * Sq, h * Dh:(h + 1) * Dh]
            for slot in folds[r]:
                fold_chunk(slot)

            for rdma in rdmas:
                rdma.wait()

        fold_chunk(7)

        attn = jnp.concatenate(
            [jnp.concatenate([acc[b * H + h] / l_t[b * H + h]
                              for h in range(H)], axis=1)
             for b in range(B)], axis=0)
        out = jnp.dot(attn, wo_ref[...], preferred_element_type=jnp.float32)
        out_ref[...] = out.reshape(B, Sq, D)

    return pl.pallas_call(
        body,
        out_shape=jax.ShapeDtypeStruct((B, Sq, D), jnp.float32),
        in_specs=[pl.BlockSpec(memory_space=pltpu.VMEM)] * 5,
        out_specs=pl.BlockSpec(memory_space=pltpu.VMEM),
        scratch_shapes=[
            pltpu.VMEM((N_DEV, 2, B, H, Skv, Dh), jnp.float32),
            pltpu.SemaphoreType.DMA((N_DEV - 1,)),
            pltpu.SemaphoreType.DMA((N_DEV - 1,)),
        ],
        compiler_params=pltpu.CompilerParams(collective_id=0),
    )(x, Wq, Wo, K_ext, V_ext)


# device time: 75257 ns/iter; 2.4846x vs baseline; 1.1305x over previous
import jax
import jax.numpy as jnp
from jax import lax
from jax.experimental import pallas as pl
from jax.experimental.pallas import tpu as pltpu

N_DEV = 8
ARRIVAL_ORDER = (1, 3, 4, 2, 5, 7, 6)


def _nt(a, b):
    return lax.dot_general(a, b, (((1,), (1,)), ((), ())),
                           preferred_element_type=jnp.float32)


def _nn(a, b):
    return lax.dot_general(a, b, (((1,), (0,)), ((), ())),
                           preferred_element_type=jnp.float32)


def kernel(x, Wq, Wo, K_ext, V_ext):
    B, Sq, D = x.shape
    _, Skv, H, Dh = K_ext.shape
    scale = 1.0 / (Dh ** 0.5)
    BH = B * H

    def body(x_ref, wq_ref, wo_ref, k_ref, v_ref, out_ref,
             kv_ref, send_sems, recv_sems):
        my = lax.axis_index("i")

        k2 = k_ref[...].reshape(B * Skv, H * Dh)
        v2 = v_ref[...].reshape(B * Skv, H * Dh)
        for b in range(B):
            for h in range(H):
                kv_ref[0, 0, b, h] = k2[b * Skv:(b + 1) * Skv,
                                        h * Dh:(h + 1) * Dh
                                        ].astype(jnp.bfloat16)
                kv_ref[0, 1, b, h] = v2[b * Skv:(b + 1) * Skv,
                                        h * Dh:(h + 1) * Dh
                                        ].astype(jnp.bfloat16)

        barrier_sem = pltpu.get_barrier_semaphore()
        for m in range(1, N_DEV):
            pl.semaphore_signal(
                barrier_sem, inc=1,
                device_id=(jnp.bitwise_xor(my, m),),
                device_id_type=pl.DeviceIdType.MESH,
            )
        pl.semaphore_wait(barrier_sem, N_DEV - 1)

        rdmas = {}
        for m in range(1, N_DEV):
            rdma = pltpu.make_async_remote_copy(
                src_ref=kv_ref.at[0],
                dst_ref=kv_ref.at[m],
                send_sem=send_sems.at[m - 1],
                recv_sem=recv_sems.at[m - 1],
                device_id=(jnp.bitwise_xor(my, m),),
                device_id_type=pl.DeviceIdType.MESH,
            )
            rdma.start()
            rdmas[m] = rdma

        q_t = [None] * BH
        m_t = [None] * BH
        l_t = [None] * BH
        acc = [None] * BH

        def fold_chunk(slot):
            kc = kv_ref[slot, 0]
            vc = kv_ref[slot, 1]
            for i in range(BH):
                b, h = divmod(i, H)
                s = _nt(q_t[i], kc[b, h]) * scale
                if m_t[i] is None:
                    m_t[i] = jnp.max(s, axis=1, keepdims=True)
                    p = jnp.exp(s - m_t[i])
                    l_t[i] = jnp.sum(p, axis=1, keepdims=True)
                    acc[i] = _nn(p.astype(jnp.bfloat16), vc[b, h])
                else:
                    m_new = jnp.maximum(m_t[i],
                                        jnp.max(s, axis=1, keepdims=True))
                    alpha = jnp.exp(m_t[i] - m_new)
                    p = jnp.exp(s - m_new)
                    l_t[i] = l_t[i] * alpha + jnp.sum(p, axis=1,
                                                      keepdims=True)
                    acc[i] = acc[i] * alpha + _nn(p.astype(jnp.bfloat16),
                                                  vc[b, h])
                    m_t[i] = m_new

        q = jnp.dot(x_ref[...].reshape(B * Sq, D), wq_ref[...],
                    preferred_element_type=jnp.float32)
        for i in range(BH):
            b, h = divmod(i, H)
            q_t[i] = q[b * Sq:(b + 1) * Sq,
                       h * Dh:(h + 1) * Dh].astype(jnp.bfloat16)
        fold_chunk(0)

        for m in ARRIVAL_ORDER:
            rdmas[m].wait_recv()
            fold_chunk(m)

        attn = jnp.concatenate(
            [jnp.concatenate([acc[b * H + h] / l_t[b * H + h]
                              for h in range(H)], axis=1)
             for b in range(B)], axis=0)
        out = jnp.dot(attn, wo_ref[...], preferred_element_type=jnp.float32)
        out_ref[...] = out.reshape(B, Sq, D)

        for m in range(1, N_DEV):
            rdmas[m].wait_send()

    return pl.pallas_call(
        body,
        out_shape=jax.ShapeDtypeStruct((B, Sq, D), jnp.float32),
        in_specs=[pl.BlockSpec(memory_space=pltpu.VMEM)] * 5,
        out_specs=pl.BlockSpec(memory_space=pltpu.VMEM),
        scratch_shapes=[
            pltpu.VMEM((N_DEV, 2, B, H, Skv, Dh), jnp.bfloat16),
            pltpu.SemaphoreType.DMA((N_DEV - 1,)),
            pltpu.SemaphoreType.DMA((N_DEV - 1,)),
        ],
        compiler_params=pltpu.CompilerParams(collective_id=0),
    )(x, Wq, Wo, K_ext, V_ext)


# device time: 51362 ns/iter; 3.6404x vs baseline; 1.4652x over previous
import jax
import jax.numpy as jnp
from jax import lax
from jax.experimental import pallas as pl
from jax.experimental.pallas import tpu as pltpu

N_DEV = 8


def _nt(a, b):
    return lax.dot_general(a, b, (((1,), (1,)), ((), ())),
                           preferred_element_type=jnp.float32)


def _nn(a, b):
    return lax.dot_general(a, b, (((1,), (0,)), ((), ())),
                           preferred_element_type=jnp.float32)


def kernel(x, Wq, Wo, K_ext, V_ext):
    B, Sq, D = x.shape
    _, Skv, H, Dh = K_ext.shape
    scale = 1.0 / (Dh ** 0.5)
    BH = B * H

    def body(x_ref, wq_ref, wo_ref, k_ref, v_ref, out_ref,
             kv_ref, send_sems, recv_sems):
        my = lax.axis_index("i")
        px = jnp.bitwise_xor(my, 1)
        py = jnp.bitwise_xor(my, 3)
        pz = jnp.bitwise_xor(my, 4)

        k2 = k_ref[...].reshape(B * Skv, H * Dh)
        v2 = v_ref[...].reshape(B * Skv, H * Dh)
        for b in range(B):
            for h in range(H):
                kv_ref[0, 0, b, h] = k2[b * Skv:(b + 1) * Skv,
                                        h * Dh:(h + 1) * Dh
                                        ].astype(jnp.bfloat16)
                kv_ref[0, 1, b, h] = v2[b * Skv:(b + 1) * Skv,
                                        h * Dh:(h + 1) * Dh
                                        ].astype(jnp.bfloat16)

        barrier_sem = pltpu.get_barrier_semaphore()
        for p in (px, py, pz):
            pl.semaphore_signal(
                barrier_sem, inc=1,
                device_id=(p,), device_id_type=pl.DeviceIdType.MESH,
            )
        pl.semaphore_wait(barrier_sem, 3)

        transfers = {
            0: (0, 1, px), 1: (0, 2, py), 2: (0, 3, pz),
            3: (2, 4, px), 4: (3, 5, py), 5: (1, 6, pz),
            6: (5, 7, px),
        }
        rdmas = {}

        def start(t):
            src, dst, partner = transfers[t]
            rdmas[t] = pltpu.make_async_remote_copy(
                src_ref=kv_ref.at[src],
                dst_ref=kv_ref.at[dst],
                send_sem=send_sems.at[t],
                recv_sem=recv_sems.at[t],
                device_id=(partner,),
                device_id_type=pl.DeviceIdType.MESH,
            )
            rdmas[t].start()

        for t in (0, 1, 2):
            start(t)

        q_t = [None] * BH
        m_t = [None] * BH
        l_t = [None] * BH
        acc = [None] * BH

        def fold_chunk(slot):
            kc = kv_ref[slot, 0]
            vc = kv_ref[slot, 1]
            for i in range(BH):
                b, h = divmod(i, H)
                s = _nt(q_t[i], kc[b, h]) * scale
                if m_t[i] is None:
                    m_t[i] = jnp.max(s, axis=1, keepdims=True)
                    p = jnp.exp(s - m_t[i])
                    l_t[i] = jnp.sum(p, axis=1, keepdims=True)
                    acc[i] = _nn(p.astype(jnp.bfloat16), vc[b, h])
                else:
                    m_new = jnp.maximum(m_t[i],
                                        jnp.max(s, axis=1, keepdims=True))
                    alpha = jnp.exp(m_t[i] - m_new)
                    p = jnp.exp(s - m_new)
                    l_t[i] = l_t[i] * alpha + jnp.sum(p, axis=1,
                                                      keepdims=True)
                    acc[i] = acc[i] * alpha + _nn(p.astype(jnp.bfloat16),
                                                  vc[b, h])
                    m_t[i] = m_new

        q = jnp.dot(x_ref[...].reshape(B * Sq, D).astype(jnp.bfloat16),
                    wq_ref[...].astype(jnp.bfloat16),
                    preferred_element_type=jnp.float32)
        wo_bf = wo_ref[...].astype(jnp.bfloat16)
        for i in range(BH):
            b, h = divmod(i, H)
            q_t[i] = q[b * Sq:(b + 1) * Sq,
                       h * Dh:(h + 1) * Dh].astype(jnp.bfloat16)
        fold_chunk(0)

        rdmas[1].wait_recv()
        start(3)
        rdmas[2].wait_recv()
        start(4)
        rdmas[0].wait_recv()
        start(5)
        fold_chunk(1)
        fold_chunk(2)
        fold_chunk(3)

        rdmas[4].wait_recv()
        start(6)
        fold_chunk(5)
        rdmas[3].wait_recv()
        fold_chunk(4)
        rdmas[5].wait_recv()
        fold_chunk(6)
        rdmas[6].wait_recv()
        fold_chunk(7)

        attn = jnp.concatenate(
            [jnp.concatenate([(acc[b * H + h] / l_t[b * H + h]
                               ).astype(jnp.bfloat16)
                              for h in range(H)], axis=1)
             for b in range(B)], axis=0)
        out = jnp.dot(attn, wo_bf, preferred_element_type=jnp.float32)
        out_ref[...] = out.reshape(B, Sq, D)

        for t in range(7):
            rdmas[t].wait_send()

    return pl.pallas_call(
        body,
        out_shape=jax.ShapeDtypeStruct((B, Sq, D), jnp.float32),
        in_specs=[pl.BlockSpec(memory_space=pltpu.VMEM)] * 5,
        out_specs=pl.BlockSpec(memory_space=pltpu.VMEM),
        scratch_shapes=[
            pltpu.VMEM((N_DEV, 2, B, H, Skv, Dh), jnp.bfloat16),
            pltpu.SemaphoreType.DMA((7,)),
            pltpu.SemaphoreType.DMA((7,)),
        ],
        compiler_params=pltpu.CompilerParams(collective_id=0),
    )(x, Wq, Wo, K_ext, V_ext)


# device time: 47918 ns/iter; 3.9021x vs baseline; 1.0719x over previous
import jax
import jax.numpy as jnp
from jax import lax
from jax.experimental import pallas as pl
from jax.experimental.pallas import tpu as pltpu

N_DEV = 8


def _nt(a, b):
    return lax.dot_general(a, b, (((1,), (1,)), ((), ())),
                           preferred_element_type=jnp.float32)


def _nn(a, b):
    return lax.dot_general(a, b, (((1,), (0,)), ((), ())),
                           preferred_element_type=jnp.float32)


def kernel(x, Wq, Wo, K_ext, V_ext):
    B, Sq, D = x.shape
    _, Skv, H, Dh = K_ext.shape
    scale = 1.0 / (Dh ** 0.5)
    BH = B * H

    def body(x_ref, wq_ref, wo_ref, k_ref, v_ref, out_ref,
             kbuf, vbuf, k_send, k_recv, v_send, v_recv):
        my = lax.axis_index("i")
        px = jnp.bitwise_xor(my, 1)
        py = jnp.bitwise_xor(my, 3)
        pz = jnp.bitwise_xor(my, 4)

        barrier_sem = pltpu.get_barrier_semaphore()
        for p in (px, py, pz):
            pl.semaphore_signal(
                barrier_sem, inc=1,
                device_id=(p,), device_id_type=pl.DeviceIdType.MESH,
            )

        k2 = k_ref[...].reshape(B * Skv, H * Dh)
        v2 = v_ref[...].reshape(B * Skv, H * Dh)
        for b in range(B):
            for h in range(H):
                kbuf[0, b, h] = k2[b * Skv:(b + 1) * Skv,
                                   h * Dh:(h + 1) * Dh].astype(jnp.bfloat16)
        pl.semaphore_wait(barrier_sem, 3)

        transfers = {
            0: (0, 1, px), 1: (0, 2, py), 2: (0, 3, pz),
            3: (2, 4, px), 4: (3, 5, py), 5: (1, 6, pz),
            6: (5, 7, px),
        }
        k_rdmas = {}
        v_rdmas = {}

        def start(t, which):
            src, dst, partner = transfers[t]
            buf, ssem, rsem, table = (
                (kbuf, k_send, k_recv, k_rdmas) if which == "k"
                else (vbuf, v_send, v_recv, v_rdmas))
            table[t] = pltpu.make_async_remote_copy(
                src_ref=buf.at[src],
                dst_ref=buf.at[dst],
                send_sem=ssem.at[t],
                recv_sem=rsem.at[t],
                device_id=(partner,),
                device_id_type=pl.DeviceIdType.MESH,
            )
            table[t].start()

        for t in (0, 1, 2):
            start(t, "k")
        for b in range(B):
            for h in range(H):
                vbuf[0, b, h] = v2[b * Skv:(b + 1) * Skv,
                                   h * Dh:(h + 1) * Dh].astype(jnp.bfloat16)
        for t in (0, 1, 2):
            start(t, "v")

        q_t = [None] * BH
        m_t = [None] * BH
        l_t = [None] * BH
        acc = [None] * BH

        def fold_chunk(slot):
            kc = kbuf[slot]
            vc = vbuf[slot]
            for i in range(BH):
                b, h = divmod(i, H)
                s = _nt(q_t[i], kc[b, h]) * scale
                if m_t[i] is None:
                    m_t[i] = jnp.max(s, axis=1, keepdims=True)
                    p = jnp.exp(s - m_t[i])
                    l_t[i] = jnp.sum(p, axis=1, keepdims=True)
                    acc[i] = _nn(p.astype(jnp.bfloat16), vc[b, h])
                else:
                    m_new = jnp.maximum(m_t[i],
                                        jnp.max(s, axis=1, keepdims=True))
                    alpha = jnp.exp(m_t[i] - m_new)
                    p = jnp.exp(s - m_new)
                    l_t[i] = l_t[i] * alpha + jnp.sum(p, axis=1,
                                                      keepdims=True)
                    acc[i] = acc[i] * alpha + _nn(p.astype(jnp.bfloat16),
                                                  vc[b, h])
                    m_t[i] = m_new

        q = jnp.dot(x_ref[...].reshape(B * Sq, D).astype(jnp.bfloat16),
                    wq_ref[...].astype(jnp.bfloat16),
                    preferred_element_type=jnp.float32)
        wo_bf = wo_ref[...].astype(jnp.bfloat16)
        for i in range(BH):
            b, h = divmod(i, H)
            q_t[i] = q[b * Sq:(b + 1) * Sq,
                       h * Dh:(h + 1) * Dh].astype(jnp.bfloat16)
        fold_chunk(0)

        k_rdmas[1].wait_recv()
        start(3, "k")
        k_rdmas[2].wait_recv()
        start(4, "k")
        k_rdmas[0].wait_recv()
        start(5, "k")
        v_rdmas[1].wait_recv()
        start(3, "v")
        v_rdmas[2].wait_recv()
        start(4, "v")
        v_rdmas[0].wait_recv()
        start(5, "v")
        fold_chunk(1)
        fold_chunk(2)
        fold_chunk(3)

        k_rdmas[4].wait_recv()
        start(6, "k")
        v_rdmas[4].wait_recv()
        start(6, "v")
        fold_chunk(5)
        k_rdmas[3].wait_recv()
        v_rdmas[3].wait_recv()
        fold_chunk(4)
        k_rdmas[5].wait_recv()
        v_rdmas[5].wait_recv()
        fold_chunk(6)
        k_rdmas[6].wait_recv()
        v_rdmas[6].wait_recv()
        fold_chunk(7)

        attn = jnp.concatenate(
            [jnp.concatenate([(acc[b * H + h] / l_t[b * H + h]
                               ).astype(jnp.bfloat16)
                              for h in range(H)], axis=1)
             for b in range(B)], axis=0)
        out = jnp.dot(attn, wo_bf, preferred_element_type=jnp.float32)
        out_ref[...] = out.reshape(B, Sq, D)

        for t in range(7):
            k_rdmas[t].wait_send()
            v_rdmas[t].wait_send()

    return pl.pallas_call(
        body,
        out_shape=jax.ShapeDtypeStruct((B, Sq, D), jnp.float32),
        in_specs=[pl.BlockSpec(memory_space=pltpu.VMEM)] * 5,
        out_specs=pl.BlockSpec(memory_space=pltpu.VMEM),
        scratch_shapes=[
            pltpu.VMEM((N_DEV, B, H, Skv, Dh), jnp.bfloat16),
            pltpu.VMEM((N_DEV, B, H, Skv, Dh), jnp.bfloat16),
            pltpu.SemaphoreType.DMA((7,)),
            pltpu.SemaphoreType.DMA((7,)),
            pltpu.SemaphoreType.DMA((7,)),
            pltpu.SemaphoreType.DMA((7,)),
        ],
        compiler_params=pltpu.CompilerParams(collective_id=0),
    )(x, Wq, Wo, K_ext, V_ext)
